# baseline (speedup 1.0000x reference)
import jax
import jax.numpy as jnp
from jax import lax
from jax.experimental import pallas as pl
from jax.experimental.pallas import tpu as pltpu
from jax.experimental.pallas import tpu_sc as plsc

B = 2
N = 10000
F = 128
E = 160000

NT = 16
L = 16

N_PAD = 10240
EPT = E // NT
K = 80
NCHUNK = EPT // K
NPT = N_PAD // NT
RSUB = K
NSUB = NPT // RSUB
RND = 1280
NRND = N_PAD // RND


def _body(x_hbm, idx_hbm, out_hbm,
          acc_sp, xch_sp, idx_v, src_v, dst_v, rows_v, hist_v,
          gsem, ssem, zsem):
  cid = lax.axis_index("c")
  sid = lax.axis_index("s")

  zero16 = jnp.zeros((L,), jnp.float32)
  one16 = jnp.ones((L,), jnp.float32)

  def rows_init(i, _):
    for p in range(2):
      for j in range(F // L):
        rows_v[p, i, pl.ds(j * L, L)] = zero16
    return 0
  lax.fori_loop(0, RSUB, rows_init, 0)

  def hist_init(i, _):
    hist_v[pl.ds(i * L, L)] = zero16
    return 0
  lax.fori_loop(0, N_PAD // L, hist_init, 0)

  for q in range(NSUB):
    pltpu.async_copy(rows_v.at[q % 2],
                     acc_sp.at[pl.ds(sid * NPT + q * RSUB, RSUB)], zsem)

  pltpu.sync_copy(idx_hbm.at[pl.ds((cid * NT + sid) * EPT, EPT)], idx_v)

  for q in range(NSUB):
    pltpu.make_async_copy(rows_v.at[0], acc_sp.at[pl.ds(0, RSUB)],
                          zsem).wait()

  plsc.subcore_barrier()

  def unpack(c, p):
    for j in range(K // L):
      w = idx_v[pl.ds(c * K + j * L, L)]
      d = lax.shift_right_logical(w, 16)
      src_v[p, 0, pl.ds(j * L, L)] = w & 0xFFFF
      dst_v[p, 0, pl.ds(j * L, L)] = d
      plsc.addupdate_scatter(hist_v, [d], one16)

  def start_gather(p):
    pltpu.async_copy(x_hbm.at[src_v.at[p, 0]], rows_v.at[p], gsem)

  def start_scatter(p):
    pltpu.async_copy(rows_v.at[p], acc_sp.at[dst_v.at[p, 0]], ssem,
                     add=True)

  def wait_gather(p):
    pltpu.make_async_copy(x_hbm.at[pl.ds(0, K)], rows_v.at[p], gsem).wait()

  def wait_scatter(p):
    pltpu.make_async_copy(rows_v.at[p], acc_sp.at[pl.ds(0, K)], ssem).wait()

  unpack(0, 0)
  start_gather(0)
  unpack(1, 1)
  start_gather(1)
  wait_gather(0)
  start_scatter(0)
  wait_gather(1)
  start_scatter(1)

  def pipe_pair(g, _):
    for p in range(2):
      c = 2 * g + 2 + p
      wait_scatter(p)
      unpack(c, p)
      start_gather(p)
      wait_gather(p)
      start_scatter(p)
    return 0
  lax.fori_loop(0, (NCHUNK - 3) // 2, pipe_pair, 0)

  wait_scatter(0)
  unpack(NCHUNK - 1, 0)
  start_gather(0)
  wait_gather(0)
  start_scatter(0)
  wait_scatter(1)
  wait_scatter(0)

  for r in range(NRND):
    pltpu.sync_copy(hist_v.at[pl.ds(r * RND, RND)],
                    xch_sp.at[pl.ds(sid * RND, RND)])
    plsc.subcore_barrier()

    @pl.when(sid // 2 == r)
    def _(r=r):
      half = (sid % 2) * NPT

      def cnt_zero(i, _):
        hist_v[pl.ds(i * L, L)] = zero16
        return 0
      lax.fori_loop(0, NPT // L, cnt_zero, 0)

      for t in range(NT):
        pltpu.sync_copy(xch_sp.at[pl.ds(t * RND + half, NPT)],
                        hist_v.at[pl.ds(NPT, NPT)])

        def cnt_add(i, _):
          sl = pl.ds(i * L, L)
          hist_v[sl] = hist_v[sl] + hist_v[pl.ds(NPT + i * L, L)]
          return 0
        lax.fori_loop(0, NPT // L, cnt_add, 0)

      def cnt_inv(i, _):
        sl = pl.ds(i * L, L)
        hist_v[sl] = 1.0 / jnp.maximum(hist_v[sl], 1.0)
        return 0
      lax.fori_loop(0, NPT // L, cnt_inv, 0)

    plsc.subcore_barrier()

  for q in range(NSUB):
    base = sid * NPT + q * RSUB

    @pl.when(base < N)
    def _(q=q, base=base):
      pltpu.sync_copy(acc_sp.at[pl.ds(base, RSUB)], rows_v.at[0])

      def scale_grp(g, _):
        cvec = hist_v[pl.ds(q * RSUB + g * L, L)]
        for k in range(L):
          inv = cvec[k]
          for j in range(F // L):
            sl = pl.ds(j * L, L)
            rows_v[0, g * L + k, sl] = rows_v[0, g * L + k, sl] * inv
        return 0
      lax.fori_loop(0, RSUB // L, scale_grp, 0)

      pltpu.sync_copy(rows_v.at[0], out_hbm.at[pl.ds(cid * N + base, RSUB)])


@jax.jit
def _graph_layer(x2, idx_all):
  mesh = plsc.VectorSubcoreMesh(core_axis_name="c", subcore_axis_name="s")
  return pl.kernel(
      _body,
      out_type=jax.ShapeDtypeStruct((B * N, F), jnp.float32),
      mesh=mesh,
      compiler_params=pltpu.CompilerParams(needs_layout_passes=False),
      scratch_types=[
          pltpu.VMEM_SHARED((N_PAD, F), jnp.float32),
          pltpu.VMEM_SHARED((NT * RND,), jnp.float32),
          pltpu.VMEM((EPT,), jnp.int32),
          pltpu.VMEM((2, 1, K), jnp.int32),
          pltpu.VMEM((2, 1, K), jnp.int32),
          pltpu.VMEM((2, K, F), jnp.float32),
          pltpu.VMEM((N_PAD,), jnp.float32),
          pltpu.SemaphoreType.DMA,
          pltpu.SemaphoreType.DMA,
          pltpu.SemaphoreType.DMA,
      ],
  )(x2, idx_all)


def kernel(X, edge_index):
  x2 = X.reshape(B * N, F)
  src = edge_index[0]
  dst = edge_index[1]
  packed = jnp.stack([src | (dst << 16), (src + N) | (dst << 16)])
  idx_all = packed.reshape(-1)
  out2 = _graph_layer(x2, idx_all)
  return out2.reshape(B, N, F)


# speedup vs baseline: 1.1289x; 1.1289x over previous
import jax
import jax.numpy as jnp
from jax import lax
from jax.experimental import pallas as pl
from jax.experimental.pallas import tpu as pltpu
from jax.experimental.pallas import tpu_sc as plsc

B = 2
N = 10000
F = 128
E = 160000

NT = 16
L = 16

N_PAD = 10240
EPT = E // NT
K = 80
NCHUNK = EPT // K
NPT = N_PAD // NT
RSUB = K
NSUB = NPT // RSUB
RND = 2560
NRND = N_PAD // RND


def _body(x_hbm, idx_hbm, out_hbm,
          acc_sp, xch_sp, idx_v, src_v, dst_v, rows_v, hist_v,
          gsem, ssem, zsem):
  cid = lax.axis_index("c")
  sid = lax.axis_index("s")

  zero16 = jnp.zeros((L,), jnp.float32)
  one16 = jnp.ones((L,), jnp.float32)

  def rows_init(i, _):
    for p in range(2):
      for j in range(F // L):
        rows_v[p, i, pl.ds(j * L, L)] = zero16
    return 0
  lax.fori_loop(0, RSUB, rows_init, 0)

  def hist_init(i, _):
    hist_v[pl.ds(i * L, L)] = zero16
    return 0
  lax.fori_loop(0, N_PAD // L, hist_init, 0)

  for q in range(NSUB):
    pltpu.async_copy(rows_v.at[q % 2],
                     acc_sp.at[pl.ds(sid * NPT + q * RSUB, RSUB)], zsem)

  pltpu.sync_copy(idx_hbm.at[pl.ds((cid * NT + sid) * EPT, EPT)], idx_v)

  for q in range(NSUB):
    pltpu.make_async_copy(rows_v.at[0], acc_sp.at[pl.ds(0, RSUB)],
                          zsem).wait()

  plsc.subcore_barrier()

  def unpack(c, p):
    for j in range(K // L):
      w = idx_v[pl.ds(c * K + j * L, L)]
      src_v[p, 0, pl.ds(j * L, L)] = w & 0xFFFF
      dst_v[p, 0, pl.ds(j * L, L)] = lax.shift_right_logical(w, 16)

  def hist_update(c):
    for j in range(K // L):
      w = idx_v[pl.ds(c * K + j * L, L)]
      plsc.addupdate_scatter(hist_v, [lax.shift_right_logical(w, 16)], one16)

  def start_gather(p):
    pltpu.async_copy(x_hbm.at[src_v.at[p, 0]], rows_v.at[p], gsem)

  def start_scatter(p):
    pltpu.async_copy(rows_v.at[p], acc_sp.at[dst_v.at[p, 0]], ssem,
                     add=True)

  def wait_gather(p):
    pltpu.make_async_copy(x_hbm.at[pl.ds(0, K)], rows_v.at[p], gsem).wait()

  def wait_scatter(p):
    pltpu.make_async_copy(rows_v.at[p], acc_sp.at[pl.ds(0, K)], ssem).wait()

  unpack(0, 0)
  start_gather(0)
  unpack(1, 1)
  start_gather(1)
  hist_update(0)
  hist_update(1)
  wait_gather(0)
  start_scatter(0)
  wait_gather(1)
  start_scatter(1)

  def pipe_pair(g, _):
    for p in range(2):
      c = 2 * g + 2 + p
      wait_scatter(p)
      unpack(c, p)
      start_gather(p)
      hist_update(c)
      wait_gather(p)
      start_scatter(p)
    return 0
  lax.fori_loop(0, (NCHUNK - 3) // 2, pipe_pair, 0)

  wait_scatter(0)
  unpack(NCHUNK - 1, 0)
  start_gather(0)
  hist_update(NCHUNK - 1)
  wait_gather(0)
  start_scatter(0)
  wait_scatter(1)
  wait_scatter(0)

  for r in range(NRND):
    pltpu.sync_copy(hist_v.at[pl.ds(r * RND, RND)],
                    xch_sp.at[pl.ds(sid * RND, RND)])
    plsc.subcore_barrier()

    @pl.when(sid // 4 == r)
    def _(r=r):
      half = (sid % 4) * NPT

      def cnt_zero(i, _):
        hist_v[pl.ds(i * L, L)] = zero16
        return 0
      lax.fori_loop(0, NPT // L, cnt_zero, 0)

      for t in range(NT):
        pltpu.sync_copy(xch_sp.at[pl.ds(t * RND + half, NPT)],
                        hist_v.at[pl.ds(NPT, NPT)])

        def cnt_add(i, _):
          sl = pl.ds(i * L, L)
          hist_v[sl] = hist_v[sl] + hist_v[pl.ds(NPT + i * L, L)]
          return 0
        lax.fori_loop(0, NPT // L, cnt_add, 0)

      def cnt_inv(i, _):
        sl = pl.ds(i * L, L)
        hist_v[sl] = 1.0 / jnp.maximum(hist_v[sl], 1.0)
        return 0
      lax.fori_loop(0, NPT // L, cnt_inv, 0)

    plsc.subcore_barrier()

  for q in range(NSUB):
    base = sid * NPT + q * RSUB

    @pl.when(base < N)
    def _(q=q, base=base):
      pltpu.sync_copy(acc_sp.at[pl.ds(base, RSUB)], rows_v.at[0])

      def scale_grp(g, _):
        cvec = hist_v[pl.ds(q * RSUB + g * L, L)]
        for k in range(L):
          inv = cvec[k]
          for j in range(F // L):
            sl = pl.ds(j * L, L)
            rows_v[0, g * L + k, sl] = rows_v[0, g * L + k, sl] * inv
        return 0
      lax.fori_loop(0, RSUB // L, scale_grp, 0)

      pltpu.sync_copy(rows_v.at[0], out_hbm.at[pl.ds(cid * N + base, RSUB)])


@jax.jit
def _graph_layer(x2, idx_all):
  mesh = plsc.VectorSubcoreMesh(core_axis_name="c", subcore_axis_name="s")
  return pl.kernel(
      _body,
      out_type=jax.ShapeDtypeStruct((B * N, F), jnp.float32),
      mesh=mesh,
      compiler_params=pltpu.CompilerParams(needs_layout_passes=False),
      scratch_types=[
          pltpu.VMEM_SHARED((N_PAD, F), jnp.float32),
          pltpu.VMEM_SHARED((NT * RND,), jnp.float32),
          pltpu.VMEM((EPT,), jnp.int32),
          pltpu.VMEM((2, 1, K), jnp.int32),
          pltpu.VMEM((2, 1, K), jnp.int32),
          pltpu.VMEM((2, K, F), jnp.float32),
          pltpu.VMEM((N_PAD,), jnp.float32),
          pltpu.SemaphoreType.DMA,
          pltpu.SemaphoreType.DMA,
          pltpu.SemaphoreType.DMA,
      ],
  )(x2, idx_all)


def kernel(X, edge_index):
  x2 = X.reshape(B * N, F)
  src = edge_index[0]
  dst = edge_index[1]
  packed = jnp.stack([src | (dst << 16), (src + N) | (dst << 16)])
  idx_all = packed.reshape(-1)
  out2 = _graph_layer(x2, idx_all)
  return out2.reshape(B, N, F)
